# baseline (speedup 1.0000x reference)
import jax
import jax.numpy as jnp
from jax import lax
from jax.experimental import pallas as pl
from jax.experimental.pallas import tpu as pltpu

_NCHUNK = 8


def _body(x_ref, o_ref, *sems):
    N = x_ref.shape[0]
    C = N // _NCHUNK
    copies = []
    for k in range(_NCHUNK):
        cp = pltpu.make_async_copy(
            x_ref.at[pl.ds(k * C, C)], o_ref.at[pl.ds(k * C, C)], sems[k]
        )
        cp.start()
        copies.append(cp)
    for cp in copies:
        cp.wait()


def kernel(x, atomic_numbers, shifts, scales):
    N, one, S, F = x.shape
    out = pl.pallas_call(
        _body,
        in_specs=[pl.BlockSpec(memory_space=pltpu.HBM)],
        out_specs=pl.BlockSpec(memory_space=pltpu.HBM),
        out_shape=jax.ShapeDtypeStruct((N, one, S, F), x.dtype),
        scratch_shapes=[pltpu.SemaphoreType.DMA] * _NCHUNK,
    )(x)
    return out


# speedup vs baseline: 5.2856x; 5.2856x over previous
import jax
import jax.numpy as jnp
from jax import lax
from jax.experimental import pallas as pl
from jax.experimental.pallas import tpu as pltpu
from jax.experimental.pallas import tpu_sc as plsc

_T = 40
_NW = 32
_F = 128
_SPH = 9


def _sc_kernel(x, atomic_numbers, shifts, scales):
    N = x.shape[0]
    nt = N // _T
    npairs = (nt + 2 * _NW - 1) // (2 * _NW)
    mesh = plsc.VectorSubcoreMesh(core_axis_name="c", subcore_axis_name="s")

    def body(x_hbm, an_hbm, sh_hbm, sc_hbm, o_hbm, xbuf, anbuf, scbuf, shbuf,
             sem_x0, sem_x1, sem_a0, sem_a1, sem_g0, sem_g1, sem_o0, sem_o1):
        wid = lax.axis_index("s") * 2 + lax.axis_index("c")
        sems = ((sem_x0, sem_a0, sem_g0, sem_o0),
                (sem_x1, sem_a1, sem_g1, sem_o1))

        def handle(j, slot):
            t = wid + _NW * j
            sx, sa, sg, so = sems[slot]

            tprev = t - 2 * _NW

            @pl.when(tprev >= 0)
            def _():
                base_p = tprev * _T
                pltpu.make_async_copy(
                    xbuf.at[slot], o_hbm.at[pl.ds(base_p, _T)], so).wait()

            @pl.when(t < nt)
            def _():
                base = t * _T
                cp_x = pltpu.make_async_copy(
                    x_hbm.at[pl.ds(base, _T)], xbuf.at[slot], sx)
                cp_a = pltpu.make_async_copy(
                    an_hbm.at[pl.ds(base, _T)], anbuf.at[slot], sa)
                cp_x.start()
                cp_a.start()
                cp_a.wait()
                cp_s = pltpu.make_async_copy(
                    sc_hbm.at[anbuf.at[slot]], scbuf.at[slot], sg)
                cp_h = pltpu.make_async_copy(
                    sh_hbm.at[anbuf.at[slot]], shbuf.at[slot], sg)
                cp_s.start()
                cp_h.start()
                cp_x.wait()
                cp_s.wait()
                cp_h.wait()

                def upd(i, carry):
                    for v in range(_F // 16):
                        d = pl.ds(v * 16, 16)
                        xv = xbuf[slot, i, d]
                        sv = jnp.abs(scbuf[slot, i, d])
                        hv = shbuf[slot, i, d]
                        xbuf[slot, i, d] = xv * sv + hv
                    return carry

                lax.fori_loop(0, _T, upd, 0)
                pltpu.make_async_copy(
                    xbuf.at[slot], o_hbm.at[pl.ds(base, _T)], so).start()

        def pair(jj, carry):
            handle(2 * jj, 0)
            handle(2 * jj + 1, 1)
            return carry

        lax.fori_loop(0, npairs, pair, 0)

        for p in (2 * npairs - 2, 2 * npairs - 1):
            t = wid + _NW * p
            slot = p % 2
            so = sems[slot][3]

            @pl.when(t < nt)
            def _():
                pltpu.make_async_copy(
                    xbuf.at[slot], o_hbm.at[pl.ds(t * _T, _T)], so).wait()

    return pl.kernel(
        body,
        mesh=mesh,
        out_type=jax.ShapeDtypeStruct(x.shape, x.dtype),
        scratch_types=[
            pltpu.VMEM((2, _T, _SPH * _F), jnp.float32),
            pltpu.VMEM((2, _T), jnp.int32),
            pltpu.VMEM((2, _T, _F), jnp.float32),
            pltpu.VMEM((2, _T, _F), jnp.float32),
        ] + [pltpu.SemaphoreType.DMA] * 8,
    )(x, atomic_numbers, shifts, scales)


def kernel(x, atomic_numbers, shifts, scales):
    N, one, S, F = x.shape
    x2 = x.reshape(N, S * F)
    out2 = _sc_kernel(x2, atomic_numbers, shifts, scales)
    return out2.reshape(N, one, S, F)


# speedup vs baseline: 10.0949x; 1.9099x over previous
import jax
import jax.numpy as jnp
from jax import lax
from jax.experimental import pallas as pl
from jax.experimental.pallas import tpu as pltpu
from jax.experimental.pallas import tpu_sc as plsc

_T = 16
_NW = 32
_F = 128
_SPH = 9


def _sc_kernel(x, atomic_numbers, shifts, scales):
    N = x.shape[0]
    nt = N // _T
    npairs = (nt + 2 * _NW - 1) // (2 * _NW)
    mesh = plsc.VectorSubcoreMesh(core_axis_name="c", subcore_axis_name="s")

    def body(x_hbm, an_hbm, sh_hbm, sc_hbm, o_hbm, xbuf, anbuf, scbuf, shbuf,
             sem_x0, sem_x1, sem_a0, sem_a1, sem_g0, sem_g1, sem_o0, sem_o1):
        wid = lax.axis_index("s") * 2 + lax.axis_index("c")
        sems = ((sem_x0, sem_a0, sem_g0, sem_o0),
                (sem_x1, sem_a1, sem_g1, sem_o1))

        def handle(j, slot):
            t = wid + _NW * j
            sx, sa, sg, so = sems[slot]

            tprev = t - 2 * _NW

            @pl.when(tprev >= 0)
            def _():
                base_p = tprev * _T
                pltpu.make_async_copy(
                    xbuf.at[slot], o_hbm.at[pl.ds(base_p, _T)], so).wait()

            @pl.when(t < nt)
            def _():
                base = t * _T
                cp_x = pltpu.make_async_copy(
                    x_hbm.at[pl.ds(base, _T)], xbuf.at[slot], sx)
                cp_a = pltpu.make_async_copy(
                    an_hbm.at[pl.ds(base, _T)], anbuf.at[slot], sa)
                cp_x.start()
                cp_a.start()
                cp_a.wait()
                cp_s = pltpu.make_async_copy(
                    sc_hbm.at[anbuf.at[slot]], scbuf.at[slot], sg)
                cp_h = pltpu.make_async_copy(
                    sh_hbm.at[anbuf.at[slot]], shbuf.at[slot], sg)
                cp_s.start()
                cp_h.start()
                cp_x.wait()
                cp_s.wait()
                cp_h.wait()

                def upd(i, carry):
                    for v in range(_F // 16):
                        d = pl.ds(v * 16, 16)
                        xv = xbuf[slot, i, 0, 0, d]
                        sv = jnp.abs(scbuf[slot, i, d])
                        hv = shbuf[slot, i, d]
                        xbuf[slot, i, 0, 0, d] = xv * sv + hv
                    return carry

                lax.fori_loop(0, _T, upd, 0)
                pltpu.make_async_copy(
                    xbuf.at[slot], o_hbm.at[pl.ds(base, _T)], so).start()

        def pair(jj, carry):
            handle(2 * jj, 0)
            handle(2 * jj + 1, 1)
            return carry

        lax.fori_loop(0, npairs, pair, 0)

        for p in (2 * npairs - 2, 2 * npairs - 1):
            t = wid + _NW * p
            slot = p % 2
            so = sems[slot][3]

            @pl.when(t < nt)
            def _():
                pltpu.make_async_copy(
                    xbuf.at[slot], o_hbm.at[pl.ds(t * _T, _T)], so).wait()

    return pl.kernel(
        body,
        mesh=mesh,
        out_type=jax.ShapeDtypeStruct(x.shape, x.dtype),
        scratch_types=[
            pltpu.VMEM((2, _T, 1, _SPH, _F), jnp.float32),
            pltpu.VMEM((2, _T), jnp.int32),
            pltpu.VMEM((2, _T, _F), jnp.float32),
            pltpu.VMEM((2, _T, _F), jnp.float32),
        ] + [pltpu.SemaphoreType.DMA] * 8,
    )(x, atomic_numbers, shifts, scales)


def kernel(x, atomic_numbers, shifts, scales):
    return _sc_kernel(x, atomic_numbers, shifts, scales)
